# baseline (speedup 1.0000x reference)
import functools

import jax
import jax.numpy as jnp
from jax import lax
from jax.experimental import pallas as pl
from jax.experimental.pallas import tpu as pltpu
from jax.experimental.pallas import tpu_sc as plsc

NU = 1000000
NI = 1000000
D = 32
B = 16384
R = 128
PACK = R // D

_NC, _NS = 2, 16
_NW = _NC * _NS
_BPW = B // _NW
_CH = 128
_NCHUNK = _BPW // _CH


def _sc_gather_body(ug_hbm, ui_hbm, um_hbm, im_hbm, urow_hbm, irow_hbm,
                    out_ug, out_ui, out_um, out_im,
                    idx_u, idx_i, rows, sem):
    wid = lax.axis_index("s") * _NC + lax.axis_index("c")
    base = wid * _BPW
    crow = wid * _NCHUNK

    pltpu.sync_copy(urow_hbm.at[pl.ds(crow, _NCHUNK)], idx_u)
    pltpu.sync_copy(irow_hbm.at[pl.ds(crow, _NCHUNK)], idx_i)

    tables = ((ug_hbm, idx_u, out_ug), (ui_hbm, idx_i, out_ui),
              (um_hbm, idx_u, out_um), (im_hbm, idx_i, out_im))
    _NBUF = 4
    nitems = 4 * _NCHUNK
    gathers = [None] * nitems

    def fire(n):
        tab, idx, _ = tables[n // _NCHUNK]
        j = n % _NCHUNK
        gathers[n] = pltpu.async_copy(tab.at[idx.at[j]], rows.at[n % _NBUF],
                                      sem)

    def retire(n):
        _, _, out = tables[n // _NCHUNK]
        j = n % _NCHUNK
        gathers[n].wait()
        pltpu.sync_copy(rows.at[n % _NBUF],
                        out.at[pl.ds(base + j * _CH, _CH)])

    for n in range(nitems):
        if n >= _NBUF:
            retire(n - _NBUF)
        fire(n)
    for n in range(nitems - _NBUF, nitems):
        retire(n)


@functools.lru_cache(maxsize=1)
def _sc_gather():
    mesh = plsc.VectorSubcoreMesh(core_axis_name="c", subcore_axis_name="s",
                                  num_cores=_NC, num_subcores=_NS)
    return pl.kernel(
        _sc_gather_body,
        out_type=[jax.ShapeDtypeStruct((B, R), jnp.float32) for _ in range(4)],
        mesh=mesh,
        scratch_types=[
            pltpu.VMEM((_NCHUNK, _CH), jnp.int32),
            pltpu.VMEM((_NCHUNK, _CH), jnp.int32),
            pltpu.VMEM((4, _CH, R), jnp.float32),
            pltpu.SemaphoreType.DMA,
        ],
    )


def _quarter(g, q):
    acc = g[:, 0 * D:1 * D]
    for k in range(1, PACK):
        acc = jnp.where(q == k, g[:, k * D:(k + 1) * D], acc)
    return acc


def _tc_body(ug, ui, um, im, uq, iq, w0t, b0, w1t, b1, w2t, b2, w3t, b3,
             wpg, wpm, bp, out):
    uqv = uq[...]
    iqv = iq[...]
    g_u = _quarter(ug[...], uqv)
    g_i = _quarter(ui[...], iqv)
    m_u = _quarter(um[...], uqv)
    m_i = _quarter(im[...], iqv)
    gmf = g_u * g_i
    w0 = w0t[...]
    h = jnp.maximum(m_u @ w0[:D] + m_i @ w0[D:] + b0[...], 0.0)
    h = jnp.maximum(h @ w1t[...] + b1[...], 0.0)
    h = jnp.maximum(h @ w2t[...] + b2[...], 0.0)
    h = jnp.maximum(h @ w3t[...] + b3[...], 0.0)
    p = gmf @ wpg[...] + h @ wpm[...] + bp[...]
    out[...] = 1.0 / (1.0 + jnp.exp(-p))


def kernel(user_indices, item_indices, embed_user_gmf, embed_item_gmf,
           embed_user_mlp, embed_item_mlp, W0, b0, W1, b1, W2, b2, W3, b3,
           Wp, bp):
    uidx = user_indices.astype(jnp.int32)
    iidx = item_indices.astype(jnp.int32)
    urow = (uidx // PACK).reshape(B // _CH, _CH)
    irow = (iidx // PACK).reshape(B // _CH, _CH)
    uq = (uidx % PACK).reshape(B, 1)
    iq = (iidx % PACK).reshape(B, 1)

    tabs = [t.reshape(NU * D // R, R) for t in
            (embed_user_gmf, embed_item_gmf, embed_user_mlp, embed_item_mlp)]

    ug, ui, um, im = _sc_gather()(*tabs, urow, irow)

    blk = 2048
    grid = B // blk
    batch_spec = pl.BlockSpec((blk, R), lambda i: (i, 0))
    q_spec = pl.BlockSpec((blk, 1), lambda i: (i, 0))

    def full(shape):
        return pl.BlockSpec(shape, lambda i: tuple(0 for _ in shape))

    w0t = W0.T
    w1t = W1.T
    w2t = W2.T
    w3t = W3.T
    wpg = Wp[:, :D].T
    wpm = Wp[:, D:].T

    out = pl.pallas_call(
        _tc_body,
        grid=(grid,),
        in_specs=[
            batch_spec, batch_spec, batch_spec, batch_spec,
            q_spec, q_spec,
            full((2 * D, 2 * D)), full((1, 2 * D)),
            full((2 * D, 32)), full((1, 32)),
            full((32, 16)), full((1, 16)),
            full((16, 8)), full((1, 8)),
            full((D, 1)), full((8, 1)), full((1, 1)),
        ],
        out_specs=pl.BlockSpec((blk, 1), lambda i: (i, 0)),
        out_shape=jax.ShapeDtypeStruct((B, 1), jnp.float32),
    )(ug, ui, um, im, uq, iq,
      w0t, b0.reshape(1, -1), w1t, b1.reshape(1, -1), w2t, b2.reshape(1, -1),
      w3t, b3.reshape(1, -1), wpg, wpm, bp.reshape(1, 1))

    return out.reshape(B)


# speedup vs baseline: 3.3819x; 3.3819x over previous
import functools

import jax
import jax.numpy as jnp
from jax import lax
from jax.experimental import pallas as pl
from jax.experimental.pallas import tpu as pltpu
from jax.experimental.pallas import tpu_sc as plsc

NU = 1000000
NI = 1000000
D = 32
B = 16384
R = 128

_NC, _NS = 2, 16
_NW = _NC * _NS
_BPW = B // _NW
_G = 16
_NGRP = _BPW // _G


def _sc_body(ug_hbm, ui_hbm, um_hbm, im_hbm, uidx_hbm, iidx_hbm, out,
             uvals, ivals, big, rows, sem):
    wid = lax.axis_index("s") * _NC + lax.axis_index("c")
    base = wid * _BPW

    pltpu.sync_copy(uidx_hbm.at[pl.ds(base, _BPW)], uvals)
    pltpu.sync_copy(iidx_hbm.at[pl.ds(base, _BPW)], ivals)

    ei = lax.iota(jnp.int32, _G)

    def group(g, carry):
        uvec = uvals[pl.ds(g * _G, _G)]
        ivec = ivals[pl.ds(g * _G, _G)]
        lu = jnp.bitwise_and(uvec, 127)
        li = jnp.bitwise_and(ivec, 127)
        tables = ((ug_hbm, uvec, lu), (ui_hbm, ivec, li),
                  (um_hbm, uvec, lu), (im_hbm, ivec, li))
        for t, (tab, vec, lvec) in enumerate(tables):
            cps = []
            for e in range(_G):
                off = pl.multiple_of((vec[e] >> 7) * 128, 128)
                cps.append(pltpu.async_copy(
                    tab.at[:, pl.ds(off, 128)],
                    big.at[pl.ds(e * D, D)], sem))
            for cp in cps:
                cp.wait()
            for d in range(D):
                dsp = jnp.full((_G,), d, jnp.int32)
                vals = plsc.load_gather(big, [ei * D + dsp, lvec])
                plsc.store_scatter(rows, [ei, dsp + t * D], vals)
        pltpu.sync_copy(rows, out.at[pl.ds(base + g * _G, _G)])
        return carry

    lax.fori_loop(0, _NGRP, group, 0)


@functools.lru_cache(maxsize=1)
def _sc_gather():
    mesh = plsc.VectorSubcoreMesh(core_axis_name="c", subcore_axis_name="s",
                                  num_cores=_NC, num_subcores=_NS)
    return pl.kernel(
        _sc_body,
        out_type=jax.ShapeDtypeStruct((B, R), jnp.float32),
        mesh=mesh,
        scratch_types=[
            pltpu.VMEM((_BPW,), jnp.int32),
            pltpu.VMEM((_BPW,), jnp.int32),
            pltpu.VMEM((_G * D, 128), jnp.float32),
            pltpu.VMEM((_G, R), jnp.float32),
            pltpu.SemaphoreType.DMA,
        ],
        compiler_params=pltpu.CompilerParams(needs_layout_passes=False),
    )


def _tc_body(x, w0t, b0, w1t, b1, w2t, b2, w3t, b3, wpg, wpm, bp, out):
    xv = x[...]
    g_u = xv[:, 0 * D:1 * D]
    g_i = xv[:, 1 * D:2 * D]
    m_u = xv[:, 2 * D:3 * D]
    m_i = xv[:, 3 * D:4 * D]
    gmf = g_u * g_i
    w0 = w0t[...]
    h = jnp.maximum(m_u @ w0[:D] + m_i @ w0[D:] + b0[...], 0.0)
    h = jnp.maximum(h @ w1t[...] + b1[...], 0.0)
    h = jnp.maximum(h @ w2t[...] + b2[...], 0.0)
    h = jnp.maximum(h @ w3t[...] + b3[...], 0.0)
    p = gmf @ wpg[...] + h @ wpm[...] + bp[...]
    out[...] = 1.0 / (1.0 + jnp.exp(-p))


def kernel(user_indices, item_indices, embed_user_gmf, embed_item_gmf,
           embed_user_mlp, embed_item_mlp, W0, b0, W1, b1, W2, b2, W3, b3,
           Wp, bp):
    uidx = user_indices.astype(jnp.int32)
    iidx = item_indices.astype(jnp.int32)

    packed = _sc_gather()(embed_user_gmf.T, embed_item_gmf.T,
                          embed_user_mlp.T, embed_item_mlp.T, uidx, iidx)

    blk = 2048
    grid = B // blk

    def full(shape):
        return pl.BlockSpec(shape, lambda i: tuple(0 for _ in shape))

    w0t = W0.T
    w1t = W1.T
    w2t = W2.T
    w3t = W3.T
    wpg = Wp[:, :D].T
    wpm = Wp[:, D:].T

    out = pl.pallas_call(
        _tc_body,
        grid=(grid,),
        in_specs=[
            pl.BlockSpec((blk, R), lambda i: (i, 0)),
            full((2 * D, 2 * D)), full((1, 2 * D)),
            full((2 * D, 32)), full((1, 32)),
            full((32, 16)), full((1, 16)),
            full((16, 8)), full((1, 8)),
            full((D, 1)), full((8, 1)), full((1, 1)),
        ],
        out_specs=pl.BlockSpec((blk, 1), lambda i: (i, 0)),
        out_shape=jax.ShapeDtypeStruct((B, 1), jnp.float32),
    )(packed,
      w0t, b0.reshape(1, -1), w1t, b1.reshape(1, -1), w2t, b2.reshape(1, -1),
      w3t, b3.reshape(1, -1), wpg, wpm, bp.reshape(1, 1))

    return out.reshape(B)


# speedup vs baseline: 3.5313x; 1.0442x over previous
import functools

import jax
import jax.numpy as jnp
from jax import lax
from jax.experimental import pallas as pl
from jax.experimental.pallas import tpu as pltpu
from jax.experimental.pallas import tpu_sc as plsc

NU = 1000000
NI = 1000000
D = 32
B = 16384
R = 128

_NC, _NS = 2, 16
_NW = _NC * _NS
_BPW = B // _NW
_G = 8
_NGRP = _BPW // _G
_CMAX = (NU - 1) >> 7


def _sc_body(ug_hbm, ui_hbm, um_hbm, im_hbm, uidx_hbm, iidx_hbm, out,
             uvals, ivals, big0, big1, rows, sem0, sem1):
    wid = lax.axis_index("s") * _NC + lax.axis_index("c")
    base = wid * _BPW

    pltpu.sync_copy(uidx_hbm.at[pl.ds(base, _BPW)], uvals.at[pl.ds(0, _BPW)])
    pltpu.sync_copy(iidx_hbm.at[pl.ds(base, _BPW)], ivals.at[pl.ds(0, _BPW)])

    ei = lax.iota(jnp.int32, 16)
    eic = jnp.minimum(ei, _G - 1)
    msk = ei < _G

    def fire(tab, vec, buf, sem):
        for e in range(_G):
            c = jnp.maximum(jnp.minimum(vec[e] >> 7, _CMAX), 0)
            off = pl.multiple_of(c * 128, 128)
            pltpu.async_copy(tab.at[:, pl.ds(off, 128)],
                             buf.at[pl.ds(e * D, D)], sem)

    def wait(buf, sem):
        for e in range(_G):
            pltpu.make_async_copy(ug_hbm.at[:, pl.ds(0, 128)],
                                  buf.at[pl.ds(e * D, D)], sem).wait()

    def extract(buf, lvec, t):
        for d in range(D):
            dsp = jnp.full((16,), d, jnp.int32)
            vals = plsc.load_gather(buf, [eic * D + dsp, lvec])
            plsc.store_scatter(rows, [ei, dsp + t * D], vals, mask=msk)

    fire(ug_hbm, uvals[pl.ds(0, 16)], big0, sem0)

    def group(g, carry):
        uvec = uvals[pl.ds(g * _G, 16)]
        ivec = ivals[pl.ds(g * _G, 16)]
        unext = uvals[pl.ds(g * _G + _G, 16)]
        lu = jnp.bitwise_and(uvec, 127)
        li = jnp.bitwise_and(ivec, 127)
        fire(ui_hbm, ivec, big1, sem1)
        wait(big0, sem0)
        extract(big0, lu, 0)
        fire(um_hbm, uvec, big0, sem0)
        wait(big1, sem1)
        extract(big1, li, 1)
        fire(im_hbm, ivec, big1, sem1)
        wait(big0, sem0)
        extract(big0, lu, 2)
        fire(ug_hbm, unext, big0, sem0)
        wait(big1, sem1)
        extract(big1, li, 3)
        pltpu.sync_copy(rows.at[pl.ds(0, _G)],
                        out.at[pl.ds(base + g * _G, _G)])
        return carry

    lax.fori_loop(0, _NGRP, group, 0)
    wait(big0, sem0)


@functools.lru_cache(maxsize=1)
def _sc_gather():
    mesh = plsc.VectorSubcoreMesh(core_axis_name="c", subcore_axis_name="s",
                                  num_cores=_NC, num_subcores=_NS)
    return pl.kernel(
        _sc_body,
        out_type=jax.ShapeDtypeStruct((B, R), jnp.float32),
        mesh=mesh,
        scratch_types=[
            pltpu.VMEM((_BPW + 16,), jnp.int32),
            pltpu.VMEM((_BPW + 16,), jnp.int32),
            pltpu.VMEM((_G * D, 128), jnp.float32),
            pltpu.VMEM((_G * D, 128), jnp.float32),
            pltpu.VMEM((16, R), jnp.float32),
            pltpu.SemaphoreType.DMA,
            pltpu.SemaphoreType.DMA,
        ],
        compiler_params=pltpu.CompilerParams(needs_layout_passes=False),
    )


def _tc_body(x, w0t, b0, w1t, b1, w2t, b2, w3t, b3, wpg, wpm, bp, out):
    xv = x[...]
    g_u = xv[:, 0 * D:1 * D]
    g_i = xv[:, 1 * D:2 * D]
    m_u = xv[:, 2 * D:3 * D]
    m_i = xv[:, 3 * D:4 * D]
    gmf = g_u * g_i
    w0 = w0t[...]
    h = jnp.maximum(m_u @ w0[:D] + m_i @ w0[D:] + b0[...], 0.0)
    h = jnp.maximum(h @ w1t[...] + b1[...], 0.0)
    h = jnp.maximum(h @ w2t[...] + b2[...], 0.0)
    h = jnp.maximum(h @ w3t[...] + b3[...], 0.0)
    p = gmf @ wpg[...] + h @ wpm[...] + bp[...]
    out[...] = 1.0 / (1.0 + jnp.exp(-p))


def kernel(user_indices, item_indices, embed_user_gmf, embed_item_gmf,
           embed_user_mlp, embed_item_mlp, W0, b0, W1, b1, W2, b2, W3, b3,
           Wp, bp):
    uidx = user_indices.astype(jnp.int32)
    iidx = item_indices.astype(jnp.int32)

    packed = _sc_gather()(embed_user_gmf.T, embed_item_gmf.T,
                          embed_user_mlp.T, embed_item_mlp.T, uidx, iidx)

    blk = 2048
    grid = B // blk

    def full(shape):
        return pl.BlockSpec(shape, lambda i: tuple(0 for _ in shape))

    w0t = W0.T
    w1t = W1.T
    w2t = W2.T
    w3t = W3.T
    wpg = Wp[:, :D].T
    wpm = Wp[:, D:].T

    out = pl.pallas_call(
        _tc_body,
        grid=(grid,),
        in_specs=[
            pl.BlockSpec((blk, R), lambda i: (i, 0)),
            full((2 * D, 2 * D)), full((1, 2 * D)),
            full((2 * D, 32)), full((1, 32)),
            full((32, 16)), full((1, 16)),
            full((16, 8)), full((1, 8)),
            full((D, 1)), full((8, 1)), full((1, 1)),
        ],
        out_specs=pl.BlockSpec((blk, 1), lambda i: (i, 0)),
        out_shape=jax.ShapeDtypeStruct((B, 1), jnp.float32),
    )(packed,
      w0t, b0.reshape(1, -1), w1t, b1.reshape(1, -1), w2t, b2.reshape(1, -1),
      w3t, b3.reshape(1, -1), wpg, wpm, bp.reshape(1, 1))

    return out.reshape(B)
